# baseline (speedup 1.0000x reference)
import jax
import jax.numpy as jnp
from jax import lax
from jax.experimental import pallas as pl
from jax.experimental.pallas import tpu as pltpu
from jax.experimental.pallas import tpu_sc as plsc

NUM_CORES = 2
NUM_SUBCORES = 16
LANES = 16
NW = NUM_CORES * NUM_SUBCORES

NUM_ROWS = 1000000
BATCH = 16384
EMBED_K = 64
BPW = BATCH // NW

TBLK = 8192


def _tc_transpose_body(src_ref, dst_ref):
    dst_ref[...] = src_ref[...].T


def _transpose_tc(GT):
    return pl.pallas_call(
        _tc_transpose_body,
        grid=(pl.cdiv(NUM_ROWS, TBLK),),
        in_specs=[pl.BlockSpec((EMBED_K, TBLK), lambda j: (0, j))],
        out_specs=pl.BlockSpec((TBLK, EMBED_K), lambda j: (j, 0)),
        out_shape=jax.ShapeDtypeStruct((NUM_ROWS, EMBED_K), jnp.float32),
    )(GT)


def _sc_body(users_h, items_h, gu_h, gi_h, xui_h, gamma_u_h, gamma_i_h,
             idx_u, idx_i, rows_u, rows_i, xui_v,
             sem_u, sem_i, sem_ou, sem_oi):
    wid = lax.axis_index("s") * NUM_CORES + lax.axis_index("c")
    base = wid * BPW

    pltpu.sync_copy(users_h.at[pl.ds(base, BPW)], idx_u)
    pltpu.sync_copy(items_h.at[pl.ds(base, BPW)], idx_i)

    cu = pltpu.async_copy(gu_h.at[idx_u], rows_u, sem_u)
    ci = pltpu.async_copy(gi_h.at[idx_i], rows_i, sem_i)
    cu.wait()
    ci.wait()

    ou = pltpu.async_copy(rows_u, gamma_u_h.at[pl.ds(base, BPW)], sem_ou)
    oi = pltpu.async_copy(rows_i, gamma_i_h.at[pl.ds(base, BPW)], sem_oi)

    lane = lax.iota(jnp.int32, LANES)

    def dot(g, _):
        acc = jnp.zeros((LANES,), jnp.float32)
        row16 = g * LANES + lane
        col = jnp.zeros((LANES,), jnp.int32)
        for _k in range(EMBED_K):
            uu = plsc.load_gather(rows_u, [row16, col])
            ii = plsc.load_gather(rows_i, [row16, col])
            acc = acc + uu * ii
            col = col + 1
        xui_v[pl.ds(g * LANES, LANES)] = acc
        return _

    lax.fori_loop(0, BPW // LANES, dot, 0)

    pltpu.sync_copy(xui_v, xui_h.at[pl.ds(base, BPW)])
    ou.wait()
    oi.wait()


def _sc_gather(users, items, Gu, Gi_rm):
    mesh = plsc.VectorSubcoreMesh(
        core_axis_name="c", subcore_axis_name="s",
        num_cores=NUM_CORES, num_subcores=NUM_SUBCORES)
    return pl.kernel(
        _sc_body,
        out_type=(
            jax.ShapeDtypeStruct((BATCH,), jnp.float32),
            jax.ShapeDtypeStruct((BATCH, EMBED_K), jnp.float32),
            jax.ShapeDtypeStruct((BATCH, EMBED_K), jnp.float32),
        ),
        mesh=mesh,
        compiler_params=pltpu.CompilerParams(
            needs_layout_passes=False, use_tc_tiling_on_sc=False),
        scratch_types=[
            pltpu.VMEM((BPW,), jnp.int32),
            pltpu.VMEM((BPW,), jnp.int32),
            pltpu.VMEM((BPW, EMBED_K), jnp.float32),
            pltpu.VMEM((BPW, EMBED_K), jnp.float32),
            pltpu.VMEM((BPW,), jnp.float32),
            pltpu.SemaphoreType.DMA,
            pltpu.SemaphoreType.DMA,
            pltpu.SemaphoreType.DMA,
            pltpu.SemaphoreType.DMA,
        ],
    )(users, items, Gu, Gi_rm)


@jax.jit
def _impl(users, items, Gu, Gi):
    Gi_rm = _transpose_tc(Gi.T)
    return _sc_gather(users, items, Gu, Gi_rm)


def kernel(users, items, Gu, Gi):
    xui, gamma_u, gamma_i = _impl(users, items, Gu, Gi)
    return (xui, gamma_u, gamma_i)


# speedup vs baseline: 1.0913x; 1.0913x over previous
import jax
import jax.numpy as jnp
from jax import lax
from jax.experimental import pallas as pl
from jax.experimental.pallas import tpu as pltpu
from jax.experimental.pallas import tpu_sc as plsc

NUM_CORES = 2
NUM_SUBCORES = 16
LANES = 16
NW = NUM_CORES * NUM_SUBCORES

NUM_ROWS = 1000000
BATCH = 16384
EMBED_K = 64
BPW = BATCH // NW

def _sc_body(users_h, items_h, gu_h, gi_h, xui_h, gamma_u_h, gamma_i_h,
             idx_u, idx_i, rows_u, rows_i, xui_v,
             sem_u, sem_i, sem_ou, sem_oi):
    wid = lax.axis_index("s") * NUM_CORES + lax.axis_index("c")
    base = wid * BPW

    pltpu.sync_copy(users_h.at[pl.ds(base, BPW)], idx_u)
    pltpu.sync_copy(items_h.at[pl.ds(base, BPW)], idx_i)

    cu = pltpu.async_copy(gu_h.at[idx_u], rows_u, sem_u)
    ci = pltpu.async_copy(gi_h.at[idx_i], rows_i, sem_i)
    cu.wait()
    ci.wait()

    ou = pltpu.async_copy(rows_u, gamma_u_h.at[pl.ds(base, BPW)], sem_ou)
    oi = pltpu.async_copy(rows_i, gamma_i_h.at[pl.ds(base, BPW)], sem_oi)

    lane = lax.iota(jnp.int32, LANES)

    def dot(g, _):
        acc = jnp.zeros((LANES,), jnp.float32)
        row16 = g * LANES + lane
        col = jnp.zeros((LANES,), jnp.int32)
        for _k in range(EMBED_K):
            uu = plsc.load_gather(rows_u, [row16, col])
            ii = plsc.load_gather(rows_i, [row16, col])
            acc = acc + uu * ii
            col = col + 1
        xui_v[pl.ds(g * LANES, LANES)] = acc
        return _

    lax.fori_loop(0, BPW // LANES, dot, 0)

    pltpu.sync_copy(xui_v, xui_h.at[pl.ds(base, BPW)])
    ou.wait()
    oi.wait()


def _sc_gather(users, items, Gu, Gi_rm):
    mesh = plsc.VectorSubcoreMesh(
        core_axis_name="c", subcore_axis_name="s",
        num_cores=NUM_CORES, num_subcores=NUM_SUBCORES)
    return pl.kernel(
        _sc_body,
        out_type=(
            jax.ShapeDtypeStruct((BATCH,), jnp.float32),
            jax.ShapeDtypeStruct((BATCH, EMBED_K), jnp.float32),
            jax.ShapeDtypeStruct((BATCH, EMBED_K), jnp.float32),
        ),
        mesh=mesh,
        compiler_params=pltpu.CompilerParams(
            needs_layout_passes=False, use_tc_tiling_on_sc=False),
        scratch_types=[
            pltpu.VMEM((BPW,), jnp.int32),
            pltpu.VMEM((BPW,), jnp.int32),
            pltpu.VMEM((BPW, EMBED_K), jnp.float32),
            pltpu.VMEM((BPW, EMBED_K), jnp.float32),
            pltpu.VMEM((BPW,), jnp.float32),
            pltpu.SemaphoreType.DMA,
            pltpu.SemaphoreType.DMA,
            pltpu.SemaphoreType.DMA,
            pltpu.SemaphoreType.DMA,
        ],
    )(users, items, Gu, Gi_rm)


@jax.jit
def _impl(users, items, Gu, Gi):
    return _sc_gather(users, items, Gu, Gi)


def kernel(users, items, Gu, Gi):
    xui, gamma_u, gamma_i = _impl(users, items, Gu, Gi)
    return (xui, gamma_u, gamma_i)
